# baseline (speedup 1.0000x reference)
import functools

import jax
import jax.numpy as jnp
from jax import lax
from jax.experimental import pallas as pl
from jax.experimental.pallas import tpu as pltpu

VOCAB_SIZE = 1000
DEPTH = VOCAB_SIZE + 1
MASK_TOKEN = -1
PAD_TOKEN = -2

BATCH = 1024
SEQ = 50
ROWS = BATCH * SEQ
RBLK = 512
GRID = ROWS // RBLK


def _ohe_tc_body(x_ref, m_ref, out_ref):
    xi = x_ref[...]
    xi = jnp.where(xi == PAD_TOKEN, VOCAB_SIZE, xi)
    m = m_ref[...]
    bad = (m == float(PAD_TOKEN)) | (m == float(MASK_TOKEN))
    keep = jnp.where(bad, 0.0, 1.0).astype(jnp.float32)
    iota = lax.broadcasted_iota(jnp.int32, (RBLK, DEPTH), 1)
    out_ref[...] = jnp.where(iota == xi, keep, 0.0)


@jax.jit
def _masked_ohe(x, mask):
    xf = x.reshape(ROWS, 1)
    mf = mask.reshape(ROWS, 1)
    out = pl.pallas_call(
        _ohe_tc_body,
        grid=(GRID,),
        in_specs=[
            pl.BlockSpec((RBLK, 1), lambda i: (i, 0)),
            pl.BlockSpec((RBLK, 1), lambda i: (i, 0)),
        ],
        out_specs=pl.BlockSpec((RBLK, DEPTH), lambda i: (i, 0)),
        out_shape=jax.ShapeDtypeStruct((ROWS, DEPTH), jnp.float32),
        compiler_params=pltpu.CompilerParams(
            dimension_semantics=("parallel",),
        ),
    )(xf, mf)
    return out.reshape(BATCH, SEQ, DEPTH)


def kernel(x, mask):
    return _masked_ohe(x.astype(jnp.int32), mask.astype(jnp.float32))


# speedup vs baseline: 1.6161x; 1.6161x over previous
import jax
import jax.numpy as jnp
from jax import lax
from jax.experimental import pallas as pl
from jax.experimental.pallas import tpu as pltpu

VOCAB_SIZE = 1000
DEPTH = VOCAB_SIZE + 1
MASK_TOKEN = -1
PAD_TOKEN = -2

BATCH = 1024
SEQ = 50
BBLK = 32
GRID = BATCH // BBLK


def _ohe_tc_body(x_ref, m_ref, out_ref):
    xi = x_ref[...]
    xi = jnp.where(xi == PAD_TOKEN, VOCAB_SIZE, xi)
    m = m_ref[...]
    bad = (m == float(PAD_TOKEN)) | (m == float(MASK_TOKEN))
    keep = jnp.where(bad, 0.0, 1.0).astype(jnp.float32)
    iota = lax.broadcasted_iota(jnp.int32, (BBLK, SEQ, DEPTH), 2)
    out_ref[...] = jnp.where(iota == xi[:, :, None], keep[:, :, None], 0.0)


@jax.jit
def _masked_ohe(x, mask):
    return pl.pallas_call(
        _ohe_tc_body,
        grid=(GRID,),
        in_specs=[
            pl.BlockSpec((BBLK, SEQ), lambda i: (i, 0)),
            pl.BlockSpec((BBLK, SEQ), lambda i: (i, 0)),
        ],
        out_specs=pl.BlockSpec((BBLK, SEQ, DEPTH), lambda i: (i, 0, 0)),
        out_shape=jax.ShapeDtypeStruct((BATCH, SEQ, DEPTH), jnp.float32),
        compiler_params=pltpu.CompilerParams(
            dimension_semantics=("parallel",),
        ),
    )(x, mask)


def kernel(x, mask):
    return _masked_ohe(x.astype(jnp.int32), mask.astype(jnp.float32))
